# Initial kernel scaffold; baseline (speedup 1.0000x reference)
import functools
import math

import jax
import jax.numpy as jnp
from jax import lax
from jax.experimental import pallas as pl
from jax.experimental.pallas import tpu as pltpu

N = 1024
D = 128
NSUB = 4
H = 2 * D
BLK = 256
GRID = N // BLK

_HIGH = lax.Precision.HIGHEST
_ENT_CONST = 0.5 + 0.5 * math.log(2.0 * math.pi)


def _body(x_ref, w1_ref, b1_ref, w2_ref, b2_ref, eps_ref,
          xout_ref, subs_ref, ent_ref):
    pid = pl.program_id(0)
    x_all = x_ref[...]
    xb = x_ref[pl.ds(pid * BLK, BLK), :]

    g = lax.dot_general(xb, x_all, (((1,), (1,)), ((), ())),
                        preferred_element_type=jnp.float32,
                        precision=_HIGH)
    n_all = jnp.sum(x_all * x_all, axis=1)[None, :]
    n_blk = jnp.sum(xb * xb, axis=1)[:, None]
    dist = n_blk + n_all - 2.0 * g

    iota_j = lax.broadcasted_iota(jnp.int32, (BLK, N), 1)

    idxs = []
    zs = []
    dcur = dist
    for _ in range(NSUB):
        mn = jnp.min(dcur, axis=1, keepdims=True)
        cand = jnp.where(dcur == mn, iota_j, jnp.int32(2 * N))
        idx = jnp.min(cand, axis=1, keepdims=True)
        idxs.append(idx)
        sel = iota_j == idx
        dcur = jnp.where(sel, jnp.float32(jnp.inf), dcur)
        onehot = sel.astype(jnp.float32)
        zk = lax.dot_general(onehot, x_all, (((1,), (0,)), ((), ())),
                             preferred_element_type=jnp.float32,
                             precision=_HIGH)
        zs.append(zk)

    subs_ref[...] = jnp.concatenate(idxs, axis=1)

    mu = (zs[0] + zs[1] + zs[2] + zs[3]) * 0.25
    mx = jnp.maximum(jnp.maximum(zs[0], zs[1]), jnp.maximum(zs[2], zs[3]))
    z = jnp.concatenate((mu, mx), axis=1)

    hdn = lax.dot_general(z, w1_ref[...], (((1,), (1,)), ((), ())),
                          preferred_element_type=jnp.float32,
                          precision=_HIGH) + b1_ref[...][None, :]
    hdn = jnp.where(hdn >= 0, hdn, 0.01 * hdn)
    z2 = lax.dot_general(hdn, w2_ref[...], (((1,), (1,)), ((), ())),
                         preferred_element_type=jnp.float32,
                         precision=_HIGH) + b2_ref[...][None, :]

    loc = z2[:, :D]
    half_log_var = z2[:, D:] * 0.5
    scale = jnp.exp(half_log_var)
    xout_ref[...] = xb + loc + scale * eps_ref[...]

    @pl.when(pid == 0)
    def _():
        ent_ref[0, 0] = 0.0
    ent_ref[0, 0] += jnp.sum(half_log_var)

    @pl.when(pid == GRID - 1)
    def _():
        ent_ref[0, 0] = _ENT_CONST + ent_ref[0, 0] / (N * D)


@jax.jit
def _run(x, W1, b1, W2, b2, eps):
    xout, subs, ent = pl.pallas_call(
        _body,
        grid=(GRID,),
        in_specs=[
            pl.BlockSpec((N, D), lambda i: (0, 0)),
            pl.BlockSpec((H, H), lambda i: (0, 0)),
            pl.BlockSpec((H,), lambda i: (0,)),
            pl.BlockSpec((H, H), lambda i: (0, 0)),
            pl.BlockSpec((H,), lambda i: (0,)),
            pl.BlockSpec((BLK, D), lambda i: (i, 0)),
        ],
        out_specs=[
            pl.BlockSpec((BLK, D), lambda i: (i, 0)),
            pl.BlockSpec((BLK, NSUB), lambda i: (i, 0)),
            pl.BlockSpec((1, 1), lambda i: (0, 0)),
        ],
        out_shape=[
            jax.ShapeDtypeStruct((N, D), jnp.float32),
            jax.ShapeDtypeStruct((N, NSUB), jnp.int32),
            jax.ShapeDtypeStruct((1, 1), jnp.float32),
        ],
    )(x, W1, b1, W2, b2, eps)
    return xout, subs, ent


def kernel(x, W1, b1, W2, b2, eps):
    xout, subs, ent = _run(x, W1, b1, W2, b2, eps)
    rows = jnp.repeat(jnp.arange(x.shape[0]), NSUB).astype(jnp.int64)
    cols = subs.reshape(-1).astype(jnp.int64)
    return (xout, ent[0, 0], rows, cols)


# speedup vs baseline: 11.6697x; 11.6697x over previous
import functools
import math

import jax
import jax.numpy as jnp
from jax import lax
from jax.experimental import pallas as pl
from jax.experimental.pallas import tpu as pltpu

N = 1024
D = 128
NSUB = 4
H = 2 * D
BLK = 256
GRID = N // BLK

_HIGH = lax.Precision.HIGHEST
_ENT_CONST = 0.5 + 0.5 * math.log(2.0 * math.pi)


def _body(x_ref, w1_ref, b1_ref, w2_ref, b2_ref, eps_ref,
          xout_ref, subs_ref, ent_ref):
    pid = pl.program_id(0)
    x_all = x_ref[...]
    xb = x_ref[pl.ds(pid * BLK, BLK), :]

    g = lax.dot_general(xb, x_all, (((1,), (1,)), ((), ())),
                        preferred_element_type=jnp.float32,
                        precision=_HIGH)
    n_all = jnp.sum(x_all * x_all, axis=1)[None, :]
    n_blk = jnp.sum(xb * xb, axis=1)[:, None]
    dist = n_blk + n_all - 2.0 * g

    iota_j = lax.broadcasted_iota(jnp.int32, (BLK, N), 1)

    idxs = []
    zs = []
    dcur = dist
    for _ in range(NSUB):
        mn = jnp.min(dcur, axis=1, keepdims=True)
        cand = jnp.where(dcur == mn, iota_j, jnp.int32(2 * N))
        idx = jnp.min(cand, axis=1, keepdims=True)
        idxs.append(idx)
        sel = iota_j == idx
        dcur = jnp.where(sel, jnp.float32(jnp.inf), dcur)
        onehot = sel.astype(jnp.float32)
        zk = lax.dot_general(onehot, x_all, (((1,), (0,)), ((), ())),
                             preferred_element_type=jnp.float32,
                             precision=_HIGH)
        zs.append(zk)

    subs_ref[...] = jnp.concatenate(idxs, axis=1)

    mu = (zs[0] + zs[1] + zs[2] + zs[3]) * 0.25
    mx = jnp.maximum(jnp.maximum(zs[0], zs[1]), jnp.maximum(zs[2], zs[3]))
    z = jnp.concatenate((mu, mx), axis=1)

    hdn = lax.dot_general(z, w1_ref[...], (((1,), (1,)), ((), ())),
                          preferred_element_type=jnp.float32,
                          precision=_HIGH) + b1_ref[...][None, :]
    hdn = jnp.where(hdn >= 0, hdn, 0.01 * hdn)
    z2 = lax.dot_general(hdn, w2_ref[...], (((1,), (1,)), ((), ())),
                         preferred_element_type=jnp.float32,
                         precision=_HIGH) + b2_ref[...][None, :]

    loc = z2[:, :D]
    half_log_var = z2[:, D:] * 0.5
    scale = jnp.exp(half_log_var)
    xout_ref[...] = xb + loc + scale * eps_ref[...]

    part = jnp.sum(half_log_var).reshape(1, 1)

    @pl.when(pid == 0)
    def _():
        ent_ref[...] = jnp.zeros((1, 1), jnp.float32)
    ent_ref[...] += part

    @pl.when(pid == GRID - 1)
    def _():
        ent_ref[...] = _ENT_CONST + ent_ref[...] / (N * D)


@jax.jit
def _run(x, W1, b1, W2, b2, eps):
    xout, subs, ent = pl.pallas_call(
        _body,
        grid=(GRID,),
        in_specs=[
            pl.BlockSpec((N, D), lambda i: (0, 0)),
            pl.BlockSpec((H, H), lambda i: (0, 0)),
            pl.BlockSpec((H,), lambda i: (0,)),
            pl.BlockSpec((H, H), lambda i: (0, 0)),
            pl.BlockSpec((H,), lambda i: (0,)),
            pl.BlockSpec((BLK, D), lambda i: (i, 0)),
        ],
        out_specs=[
            pl.BlockSpec((BLK, D), lambda i: (i, 0)),
            pl.BlockSpec((BLK, NSUB), lambda i: (i, 0)),
            pl.BlockSpec((1, 1), lambda i: (0, 0)),
        ],
        out_shape=[
            jax.ShapeDtypeStruct((N, D), jnp.float32),
            jax.ShapeDtypeStruct((N, NSUB), jnp.int32),
            jax.ShapeDtypeStruct((1, 1), jnp.float32),
        ],
    )(x, W1, b1, W2, b2, eps)
    return xout, subs, ent


def kernel(x, W1, b1, W2, b2, eps):
    xout, subs, ent = _run(x, W1, b1, W2, b2, eps)
    rows = jnp.repeat(jnp.arange(x.shape[0]), NSUB).astype(jnp.int64)
    cols = subs.reshape(-1).astype(jnp.int64)
    return (xout, ent[0, 0], rows, cols)


# speedup vs baseline: 20.4855x; 1.7554x over previous
import functools
import math

import jax
import jax.numpy as jnp
from jax import lax
from jax.experimental import pallas as pl
from jax.experimental.pallas import tpu as pltpu

N = 1024
D = 128
NSUB = 4
H = 2 * D
BLK = 256
GRID = N // BLK

_HIGH = lax.Precision.HIGHEST
_ENT_CONST = 0.5 + 0.5 * math.log(2.0 * math.pi)


def _body(x_ref, w1_ref, b1_ref, w2_ref, b2_ref, eps_ref,
          xout_ref, subs_ref, ent_ref):
    pid = pl.program_id(0)
    x_all = x_ref[...]
    xb = x_ref[pl.ds(pid * BLK, BLK), :]

    g = lax.dot_general(xb, x_all, (((1,), (1,)), ((), ())),
                        preferred_element_type=jnp.float32,
                        precision=_HIGH)
    n_all = jnp.sum(x_all * x_all, axis=1)[None, :]
    n_blk = jnp.sum(xb * xb, axis=1)[:, None]
    dist = n_blk + n_all - 2.0 * g

    iota_j = lax.broadcasted_iota(jnp.int32, (BLK, N), 1)

    idxs = []
    zs = []
    dcur = dist
    for _ in range(NSUB):
        mn = jnp.min(dcur, axis=1, keepdims=True)
        cand = jnp.where(dcur == mn, iota_j, jnp.int32(2 * N))
        idx = jnp.min(cand, axis=1, keepdims=True)
        idxs.append(idx)
        sel = iota_j == idx
        dcur = jnp.where(sel, jnp.float32(jnp.inf), dcur)
        onehot = sel.astype(jnp.float32)
        zk = lax.dot_general(onehot, x_all, (((1,), (0,)), ((), ())),
                             preferred_element_type=jnp.float32)
        zs.append(zk)

    subs_ref[...] = jnp.concatenate(idxs, axis=1)

    mu = (zs[0] + zs[1] + zs[2] + zs[3]) * 0.25
    mx = jnp.maximum(jnp.maximum(zs[0], zs[1]), jnp.maximum(zs[2], zs[3]))
    z = jnp.concatenate((mu, mx), axis=1)

    hdn = lax.dot_general(z, w1_ref[...], (((1,), (1,)), ((), ())),
                          preferred_element_type=jnp.float32) + b1_ref[...][None, :]
    hdn = jnp.where(hdn >= 0, hdn, 0.01 * hdn)
    z2 = lax.dot_general(hdn, w2_ref[...], (((1,), (1,)), ((), ())),
                         preferred_element_type=jnp.float32) + b2_ref[...][None, :]

    loc = z2[:, :D]
    half_log_var = z2[:, D:] * 0.5
    scale = jnp.exp(half_log_var)
    xout_ref[...] = xb + loc + scale * eps_ref[...]

    part = jnp.sum(half_log_var).reshape(1, 1)

    @pl.when(pid == 0)
    def _():
        ent_ref[...] = jnp.zeros((1, 1), jnp.float32)
    ent_ref[...] += part

    @pl.when(pid == GRID - 1)
    def _():
        ent_ref[...] = _ENT_CONST + ent_ref[...] / (N * D)


@jax.jit
def _run(x, W1, b1, W2, b2, eps):
    xout, subs, ent = pl.pallas_call(
        _body,
        grid=(GRID,),
        in_specs=[
            pl.BlockSpec((N, D), lambda i: (0, 0)),
            pl.BlockSpec((H, H), lambda i: (0, 0)),
            pl.BlockSpec((H,), lambda i: (0,)),
            pl.BlockSpec((H, H), lambda i: (0, 0)),
            pl.BlockSpec((H,), lambda i: (0,)),
            pl.BlockSpec((BLK, D), lambda i: (i, 0)),
        ],
        out_specs=[
            pl.BlockSpec((BLK, D), lambda i: (i, 0)),
            pl.BlockSpec((BLK, NSUB), lambda i: (i, 0)),
            pl.BlockSpec((1, 1), lambda i: (0, 0)),
        ],
        out_shape=[
            jax.ShapeDtypeStruct((N, D), jnp.float32),
            jax.ShapeDtypeStruct((N, NSUB), jnp.int32),
            jax.ShapeDtypeStruct((1, 1), jnp.float32),
        ],
    )(x, W1, b1, W2, b2, eps)
    return xout, subs, ent


def kernel(x, W1, b1, W2, b2, eps):
    xout, subs, ent = _run(x, W1, b1, W2, b2, eps)
    rows = jnp.repeat(jnp.arange(x.shape[0]), NSUB).astype(jnp.int64)
    cols = subs.reshape(-1).astype(jnp.int64)
    return (xout, ent[0, 0], rows, cols)
